# Initial kernel scaffold; baseline (speedup 1.0000x reference)
import jax
import jax.numpy as jnp
from jax.experimental import pallas as pl


def kernel(inputs, supports, initial_hidden_state, Wg0, bg0, Wc0, bc0, Wg1, bg1, Wc1, bc1):
    raise NotImplementedError("write your pallas kernel here")


# speedup vs baseline: 7.5572x; 7.5572x over previous
import functools

import jax
import jax.numpy as jnp
from jax.experimental import pallas as pl
from jax.experimental.pallas import tpu as pltpu

NODE = 325
BATCH = 32
HID = 64
T = 12
NMAT = 5
BB = 8


def _layer_kernel(I, x_ref, sup_ref, h0_ref, wg_ref, bg_ref, wc_ref, bc_ref,
                  out_ref, h_scr):
    t = pl.program_id(1)

    @pl.when(t == 0)
    def _():
        h_scr[...] = h0_ref[...]

    F = I + HID
    x3 = x_ref[0]
    h3 = h_scr[...]

    def dconv(s3, w_ref, b_ref):
        out_dim = w_ref.shape[2]
        xs3 = jnp.concatenate([x3, s3], axis=2)
        xs = xs3.reshape(NODE, BB * F)

        def proj(m, mat3):
            if F < 128:
                mat3 = jnp.concatenate(
                    [mat3, jnp.zeros((NODE, BB, 128 - F), jnp.float32)],
                    axis=2)
            r = mat3.reshape(NODE * BB, 128)
            return jnp.dot(r, w_ref[m], preferred_element_type=jnp.float32)

        acc = proj(0, xs3) + b_ref[...]
        m = 1
        for s in range(2):
            sup = sup_ref[s]
            x1 = jnp.dot(sup, xs, preferred_element_type=jnp.float32)
            acc = acc + proj(m, x1.reshape(NODE, BB, F))
            m += 1
            x2 = 2.0 * jnp.dot(sup, x1, preferred_element_type=jnp.float32) - xs
            acc = acc + proj(m, x2.reshape(NODE, BB, F))
            m += 1
        return acc.reshape(NODE, BB, out_dim)

    g = jax.nn.sigmoid(dconv(h3, wg_ref, bg_ref))
    r = g[:, :, :HID]
    u = g[:, :, HID:]
    c = jnp.tanh(dconv(r * h3, wc_ref, bc_ref))
    h_new = u * h3 + (1.0 - u) * c
    h_scr[...] = h_new
    out_ref[0] = h_new


def _run_layer(xseq, supports, h0, wg, bg, wc, bc, I):
    kern = functools.partial(_layer_kernel, I)
    nb = BATCH // BB
    return pl.pallas_call(
        kern,
        grid=(nb, T),
        in_specs=[
            pl.BlockSpec((1, NODE, BB, I), lambda b, t: (t, 0, b, 0)),
            pl.BlockSpec((2, NODE, NODE), lambda b, t: (0, 0, 0)),
            pl.BlockSpec((NODE, BB, HID), lambda b, t: (0, b, 0)),
            pl.BlockSpec((NMAT, 128, 2 * HID), lambda b, t: (0, 0, 0)),
            pl.BlockSpec((1, 2 * HID), lambda b, t: (0, 0)),
            pl.BlockSpec((NMAT, 128, HID), lambda b, t: (0, 0, 0)),
            pl.BlockSpec((1, HID), lambda b, t: (0, 0)),
        ],
        out_specs=pl.BlockSpec((1, NODE, BB, HID), lambda b, t: (t, 0, b, 0)),
        out_shape=jax.ShapeDtypeStruct((T, NODE, BATCH, HID), jnp.float32),
        scratch_shapes=[pltpu.VMEM((NODE, BB, HID), jnp.float32)],
        compiler_params=pltpu.CompilerParams(
            dimension_semantics=("arbitrary", "arbitrary")),
    )(xseq, supports, h0, wg, bg, wc, bc)


def _reorder_w(w, F):
    out_dim = w.shape[1]
    w = w.reshape(F, NMAT, out_dim).transpose(1, 0, 2)
    if F < 128:
        w = jnp.pad(w, ((0, 0), (0, 128 - F), (0, 0)))
    return w


def kernel(inputs, supports, initial_hidden_state,
           Wg0, bg0, Wc0, bc0, Wg1, bg1, Wc1, bc1):
    x0 = inputs.reshape(T, BATCH, NODE, 2).transpose(0, 2, 1, 3)
    h0 = initial_hidden_state.reshape(2, BATCH, NODE, HID).transpose(0, 2, 1, 3)

    out0 = _run_layer(x0, supports, h0[0],
                      _reorder_w(Wg0, 2 + HID), bg0.reshape(1, -1),
                      _reorder_w(Wc0, 2 + HID), bc0.reshape(1, -1), 2)
    out1 = _run_layer(out0, supports, h0[1],
                      _reorder_w(Wg1, HID + HID), bg1.reshape(1, -1),
                      _reorder_w(Wc1, HID + HID), bc1.reshape(1, -1), HID)

    cur = out1.transpose(0, 2, 1, 3).reshape(T, BATCH, NODE * HID)
    hfin = jnp.stack([out0[T - 1], out1[T - 1]], axis=0)
    hfin = hfin.transpose(0, 2, 1, 3).reshape(2, BATCH, NODE * HID)
    return (hfin, cur)


# speedup vs baseline: 8.7498x; 1.1578x over previous
import functools

import jax
import jax.numpy as jnp
from jax.experimental import pallas as pl
from jax.experimental.pallas import tpu as pltpu

NODE = 325
BATCH = 32
HID = 64
T = 12
NMAT = 5
BB = 16


def _layer_kernel(I, x_ref, sup_ref, h0_ref, wg_ref, bg_ref, wc_ref, bc_ref,
                  out_ref, h_scr):
    t = pl.program_id(1)

    @pl.when(t == 0)
    def _():
        h_scr[...] = h0_ref[...]

    F = I + HID
    x3 = x_ref[0]
    h3 = h_scr[...]

    def dconv(s3, w_ref, b_ref):
        out_dim = w_ref.shape[2]
        xs3 = jnp.concatenate([x3, s3], axis=2)
        xs = xs3.reshape(NODE, BB * F)

        def proj(m, mat3):
            if F < 128:
                mat3 = jnp.concatenate(
                    [mat3, jnp.zeros((NODE, BB, 128 - F), jnp.float32)],
                    axis=2)
            r = mat3.reshape(NODE * BB, 128)
            return jnp.dot(r, w_ref[m], preferred_element_type=jnp.float32)

        acc = proj(0, xs3) + b_ref[...]
        m = 1
        for s in range(2):
            sup = sup_ref[s]
            x1 = jnp.dot(sup, xs, preferred_element_type=jnp.float32)
            acc = acc + proj(m, x1.reshape(NODE, BB, F))
            m += 1
            x2 = 2.0 * jnp.dot(sup, x1, preferred_element_type=jnp.float32) - xs
            acc = acc + proj(m, x2.reshape(NODE, BB, F))
            m += 1
        return acc.reshape(NODE, BB, out_dim)

    g = jax.nn.sigmoid(dconv(h3, wg_ref, bg_ref))
    r = g[:, :, :HID]
    u = g[:, :, HID:]
    c = jnp.tanh(dconv(r * h3, wc_ref, bc_ref))
    h_new = u * h3 + (1.0 - u) * c
    h_scr[...] = h_new
    out_ref[0] = h_new


def _run_layer(xseq, supports, h0, wg, bg, wc, bc, I):
    kern = functools.partial(_layer_kernel, I)
    nb = BATCH // BB
    return pl.pallas_call(
        kern,
        grid=(nb, T),
        in_specs=[
            pl.BlockSpec((1, NODE, BB, I), lambda b, t: (t, 0, b, 0)),
            pl.BlockSpec((2, NODE, NODE), lambda b, t: (0, 0, 0)),
            pl.BlockSpec((NODE, BB, HID), lambda b, t: (0, b, 0)),
            pl.BlockSpec((NMAT, 128, 2 * HID), lambda b, t: (0, 0, 0)),
            pl.BlockSpec((1, 2 * HID), lambda b, t: (0, 0)),
            pl.BlockSpec((NMAT, 128, HID), lambda b, t: (0, 0, 0)),
            pl.BlockSpec((1, HID), lambda b, t: (0, 0)),
        ],
        out_specs=pl.BlockSpec((1, NODE, BB, HID), lambda b, t: (t, 0, b, 0)),
        out_shape=jax.ShapeDtypeStruct((T, NODE, BATCH, HID), jnp.float32),
        scratch_shapes=[pltpu.VMEM((NODE, BB, HID), jnp.float32)],
        compiler_params=pltpu.CompilerParams(
            dimension_semantics=("arbitrary", "arbitrary")),
    )(xseq, supports, h0, wg, bg, wc, bc)


def _reorder_w(w, F):
    out_dim = w.shape[1]
    w = w.reshape(F, NMAT, out_dim).transpose(1, 0, 2)
    if F < 128:
        w = jnp.pad(w, ((0, 0), (0, 128 - F), (0, 0)))
    return w


def kernel(inputs, supports, initial_hidden_state,
           Wg0, bg0, Wc0, bc0, Wg1, bg1, Wc1, bc1):
    x0 = inputs.reshape(T, BATCH, NODE, 2).transpose(0, 2, 1, 3)
    h0 = initial_hidden_state.reshape(2, BATCH, NODE, HID).transpose(0, 2, 1, 3)

    out0 = _run_layer(x0, supports, h0[0],
                      _reorder_w(Wg0, 2 + HID), bg0.reshape(1, -1),
                      _reorder_w(Wc0, 2 + HID), bc0.reshape(1, -1), 2)
    out1 = _run_layer(out0, supports, h0[1],
                      _reorder_w(Wg1, HID + HID), bg1.reshape(1, -1),
                      _reorder_w(Wc1, HID + HID), bc1.reshape(1, -1), HID)

    cur = out1.transpose(0, 2, 1, 3).reshape(T, BATCH, NODE * HID)
    hfin = jnp.stack([out0[T - 1], out1[T - 1]], axis=0)
    hfin = hfin.transpose(0, 2, 1, 3).reshape(2, BATCH, NODE * HID)
    return (hfin, cur)


# speedup vs baseline: 8.7529x; 1.0004x over previous
import jax
import jax.numpy as jnp
from jax.experimental import pallas as pl
from jax.experimental.pallas import tpu as pltpu

NODE = 325
BATCH = 32
HID = 64
T = 12
NMAT = 5
BB = 16
F = 2 * HID


def _layer_kernel(x_ref, sup_ref, h0_ref, wg_ref, bg_ref, wc_ref, bc_ref,
                  out_ref, h_scr):
    t = pl.program_id(1)

    @pl.when(t == 0)
    def _():
        h_scr[...] = h0_ref[...]

    x3 = x_ref[0]
    h3 = h_scr[...]

    def dconv(s3, w_ref, b_ref):
        out_dim = w_ref.shape[2]
        xs = jnp.concatenate([x3, s3], axis=2).reshape(NODE, BB * F)

        def proj(m, mat):
            r = mat.reshape(NODE * BB, F)
            return jnp.dot(r, w_ref[m], preferred_element_type=jnp.float32)

        acc = proj(0, xs) + b_ref[...]
        m = 1
        for s in range(2):
            sup = sup_ref[s]
            x1 = jnp.dot(sup, xs, preferred_element_type=jnp.float32)
            acc = acc + proj(m, x1)
            m += 1
            x2 = 2.0 * jnp.dot(sup, x1, preferred_element_type=jnp.float32) - xs
            acc = acc + proj(m, x2)
            m += 1
        return acc.reshape(NODE, BB, out_dim)

    g = jax.nn.sigmoid(dconv(h3, wg_ref, bg_ref))
    r = g[:, :, :HID]
    u = g[:, :, HID:]
    c = jnp.tanh(dconv(r * h3, wc_ref, bc_ref))
    h_new = u * h3 + (1.0 - u) * c
    h_scr[...] = h_new
    out_ref[0] = h_new


def _run_layer(xseq, supports, h0, wg, bg, wc, bc):
    nb = BATCH // BB
    return pl.pallas_call(
        _layer_kernel,
        grid=(nb, T),
        in_specs=[
            pl.BlockSpec((1, NODE, BB, HID), lambda b, t: (t, 0, b, 0)),
            pl.BlockSpec((2, NODE, NODE), lambda b, t: (0, 0, 0)),
            pl.BlockSpec((NODE, BB, HID), lambda b, t: (0, b, 0)),
            pl.BlockSpec((NMAT, F, 2 * HID), lambda b, t: (0, 0, 0)),
            pl.BlockSpec((1, 2 * HID), lambda b, t: (0, 0)),
            pl.BlockSpec((NMAT, F, HID), lambda b, t: (0, 0, 0)),
            pl.BlockSpec((1, HID), lambda b, t: (0, 0)),
        ],
        out_specs=pl.BlockSpec((1, NODE, BB, HID), lambda b, t: (t, 0, b, 0)),
        out_shape=jax.ShapeDtypeStruct((T, NODE, BATCH, HID), jnp.float32),
        scratch_shapes=[pltpu.VMEM((NODE, BB, HID), jnp.float32)],
        compiler_params=pltpu.CompilerParams(
            dimension_semantics=("arbitrary", "arbitrary")),
    )(xseq, supports, h0, wg, bg, wc, bc)


def _reorder_w(w, I):
    out_dim = w.shape[1]
    w = w.reshape(I + HID, NMAT, out_dim).transpose(1, 0, 2)
    if I < HID:
        w = jnp.concatenate(
            [w[:, :I], jnp.zeros((NMAT, HID - I, out_dim), w.dtype), w[:, I:]],
            axis=1)
    return w


def kernel(inputs, supports, initial_hidden_state,
           Wg0, bg0, Wc0, bc0, Wg1, bg1, Wc1, bc1):
    x0 = inputs.reshape(T, BATCH, NODE, 2).transpose(0, 2, 1, 3)
    x0 = jnp.pad(x0, ((0, 0), (0, 0), (0, 0), (0, HID - 2)))
    h0 = initial_hidden_state.reshape(2, BATCH, NODE, HID).transpose(0, 2, 1, 3)

    out0 = _run_layer(x0, supports, h0[0],
                      _reorder_w(Wg0, 2), bg0.reshape(1, -1),
                      _reorder_w(Wc0, 2), bc0.reshape(1, -1))
    out1 = _run_layer(out0, supports, h0[1],
                      _reorder_w(Wg1, HID), bg1.reshape(1, -1),
                      _reorder_w(Wc1, HID), bc1.reshape(1, -1))

    cur = out1.transpose(0, 2, 1, 3).reshape(T, BATCH, NODE * HID)
    hfin = jnp.stack([out0[T - 1], out1[T - 1]], axis=0)
    hfin = hfin.transpose(0, 2, 1, 3).reshape(2, BATCH, NODE * HID)
    return (hfin, cur)
